# baseline (speedup 1.0000x reference)
import functools

import jax
import jax.numpy as jnp
from jax import lax
from jax.experimental import pallas as pl
from jax.experimental.pallas import tpu as pltpu
from jax.experimental.pallas import tpu_sc as plsc

NC = 2
NS = 16
NW = NC * NS
L = 16

N_ROWS = 4096
D = 512
N_TRIP = 16384
STEPS = D // L

TPW = N_TRIP // NW
CH = 32
N_CHUNKS = TPW // CH


def _sc_diff_body(x_hbm, ti_hbm, tj_hbm, tk_hbm, out_hbm,
                  idx_i, idx_j, idx_k, r_i, r_j, r_k, out_v, sem):
    wid = lax.axis_index("s") * NC + lax.axis_index("c")
    base = wid * TPW

    pltpu.sync_copy(ti_hbm.at[pl.ds(base, TPW)], idx_i)
    pltpu.sync_copy(tj_hbm.at[pl.ds(base, TPW)], idx_j)
    pltpu.sync_copy(tk_hbm.at[pl.ds(base, TPW)], idx_k)

    def chunk_body(c, carry):
        off = c * CH
        h_i = pltpu.async_copy(x_hbm.at[idx_i.at[pl.ds(off, CH)]], r_i, sem)
        h_j = pltpu.async_copy(x_hbm.at[idx_j.at[pl.ds(off, CH)]], r_j, sem)
        h_k = pltpu.async_copy(x_hbm.at[idx_k.at[pl.ds(off, CH)]], r_k, sem)
        h_i.wait()
        h_j.wait()
        h_k.wait()

        def trip_body(t, tc):
            zero = jnp.zeros((L,), jnp.float32)
            acc = [[zero, zero] for _ in range(5)]
            for s in range(STEPS):
                p = s & 1
                sl = pl.ds(s * L, L)
                vi = r_i[t, sl]
                vj = r_j[t, sl]
                vk = r_k[t, sl]
                acc[0][p] = acc[0][p] + vi * vi
                acc[1][p] = acc[1][p] + vj * vj
                acc[2][p] = acc[2][p] + vk * vk
                acc[3][p] = acc[3][p] + vi * vj
                acc[4][p] = acc[4][p] + vi * vk
            s_ii = jnp.sum(acc[0][0] + acc[0][1])
            s_jj = jnp.sum(acc[1][0] + acc[1][1])
            s_kk = jnp.sum(acc[2][0] + acc[2][1])
            s_ij = jnp.sum(acc[3][0] + acc[3][1])
            s_ik = jnp.sum(acc[4][0] + acc[4][1])
            dij = jnp.maximum(s_ii + s_jj - 2.0 * s_ij, 0.0)
            dik = jnp.maximum(s_ii + s_kk - 2.0 * s_ik, 0.0)
            lanes = lax.iota(jnp.int32, L)
            plsc.store_scatter(
                out_v,
                [jnp.full((L,), off + t, jnp.int32)],
                jnp.full((L,), dij - dik, jnp.float32),
                mask=lanes == 0,
            )
            return tc

        lax.fori_loop(0, CH, trip_body, 0, unroll=False)
        return carry

    lax.fori_loop(0, N_CHUNKS, chunk_body, 0, unroll=False)
    pltpu.sync_copy(out_v, out_hbm.at[pl.ds(base, TPW)])


_sc_diffs = functools.partial(
    pl.kernel,
    out_type=jax.ShapeDtypeStruct((N_TRIP,), jnp.float32),
    mesh=plsc.VectorSubcoreMesh(
        core_axis_name="c", subcore_axis_name="s",
        num_cores=NC, num_subcores=NS),
    compiler_params=pltpu.CompilerParams(needs_layout_passes=False),
    scratch_types=[
        pltpu.VMEM((TPW,), jnp.int32),
        pltpu.VMEM((TPW,), jnp.int32),
        pltpu.VMEM((TPW,), jnp.int32),
        pltpu.VMEM((CH, D), jnp.float32),
        pltpu.VMEM((CH, D), jnp.float32),
        pltpu.VMEM((CH, D), jnp.float32),
        pltpu.VMEM((TPW,), jnp.float32),
        pltpu.SemaphoreType.DMA,
    ],
)(_sc_diff_body)


def _tc_loss_body(d_ref, o_ref):
    d = d_ref[...]
    per = jnp.log(1.0 + jnp.exp(d))
    o_ref[...] = jnp.reshape(jnp.sum(per) / float(N_TRIP), (1, 1))


def kernel(x, triplets):
    t32 = triplets.astype(jnp.int32)
    ti = t32[:, 0]
    tj = t32[:, 1]
    tk = t32[:, 2]
    diffs = _sc_diffs(x, ti, tj, tk)
    loss = pl.pallas_call(
        _tc_loss_body,
        out_shape=jax.ShapeDtypeStruct((1, 1), jnp.float32),
    )(diffs.reshape(128, 128))
    return loss.reshape(1)
